# baseline (speedup 1.0000x reference)
import functools

import jax
import jax.numpy as jnp
from jax import lax
from jax.experimental import pallas as pl
from jax.experimental.pallas import tpu as pltpu
from jax.experimental.pallas import tpu_sc as plsc

_NC = 2
_NS = 16
_NW = _NC * _NS
_CHUNK = 128
_NPHASE = 4
_NBUF = 2


def _sc_aggregate(x, src3, dst3, *, n_pad, d, k_chunks):
  p_chunks = k_chunks // _NPHASE

  def body(x_hbm, src_hbm, dst_hbm, acc_out, idx_src, idx_dst, rows, acc_sh,
           gsems, isem):
    cid = lax.axis_index("c")
    sid = lax.axis_index("s")
    wid = sid * _NC + cid
    zeros16 = jnp.zeros((16,), jnp.float32)

    pltpu.sync_copy(src_hbm.at[wid], idx_src)
    pltpu.sync_copy(dst_hbm.at[wid], idx_dst)

    def zbody(i, c):
      for j in range(d // 16):
        rows[0, i, pl.ds(j * 16, 16)] = zeros16
      return c

    lax.fori_loop(0, _CHUNK, zbody, 0)

    zpt = n_pad // _NS
    base = sid * zpt
    for t in range(zpt // _CHUNK):
      pltpu.sync_copy(rows.at[0], acc_sh.at[pl.ds(base + t * _CHUNK, _CHUNK)])
    rem = zpt % _CHUNK
    if rem:
      pltpu.sync_copy(rows.at[0, pl.ds(0, rem)],
                      acc_sh.at[pl.ds(base + zpt - rem, rem)])
    plsc.subcore_barrier()

    def ebody(j, c):
      pltpu.async_copy(x_hbm.at[idx_src.at[j]], rows.at[0],
                       gsems.at[0]).wait()
      pltpu.sync_copy(rows.at[0], acc_sh.at[idx_dst.at[j]], add=True)
      return c

    lax.fori_loop(0, k_chunks, ebody, 0)
    plsc.subcore_barrier()

    cpt = n_pad // _NS
    pltpu.sync_copy(acc_sh.at[pl.ds(sid * cpt, cpt)],
                    acc_out.at[cid, pl.ds(sid * cpt, cpt)])

  mesh = plsc.VectorSubcoreMesh(core_axis_name="c", subcore_axis_name="s")
  return pl.kernel(
      body,
      out_type=jax.ShapeDtypeStruct((_NC, n_pad, d), jnp.float32),
      mesh=mesh,
      compiler_params=pltpu.CompilerParams(needs_layout_passes=False),
      scratch_types=[
          pltpu.VMEM((k_chunks, _CHUNK), jnp.int32),
          pltpu.VMEM((k_chunks, _CHUNK), jnp.int32),
          pltpu.VMEM((1, _CHUNK, d), jnp.float32),
          pltpu.VMEM_SHARED((n_pad, d), jnp.float32),
          pltpu.SemaphoreType.DMA((_NBUF,)),
          pltpu.SemaphoreType.DMA,
      ],
  )(x, src3, dst3)


def _sc_degree(dst3, *, n_pad, k_chunks):
  half = k_chunks // 2

  def body(dst_hbm, deg_out, idx, hist, isem):
    cid = lax.axis_index("c")
    sid = lax.axis_index("s")
    wid = sid * _NC + cid
    ones16 = jnp.ones((16,), jnp.float32)
    zeros16 = jnp.zeros((16,), jnp.float32)

    pltpu.sync_copy(dst_hbm.at[wid, pl.ds(0, half)], idx.at[0])
    pltpu.async_copy(dst_hbm.at[wid, pl.ds(half, half)], idx.at[1], isem)

    def hzero(i, c):
      hist[pl.ds(i * 16, 16)] = zeros16
      return c

    lax.fori_loop(0, n_pad // 16, hzero, 0)

    def count(s):
      def cbody(t, c):
        for q in range(_CHUNK // 16):
          v = idx[s, t, pl.ds(q * 16, 16)]
          plsc.addupdate_scatter(hist, [v], ones16)
        return c

      lax.fori_loop(0, half, cbody, 0)

    count(0)
    pltpu.make_async_copy(dst_hbm.at[wid, pl.ds(0, half)], idx.at[1],
                          isem).wait()
    count(1)
    pltpu.sync_copy(hist, deg_out.at[wid])

  mesh = plsc.VectorSubcoreMesh(core_axis_name="c", subcore_axis_name="s")
  return pl.kernel(
      body,
      out_type=jax.ShapeDtypeStruct((_NW, n_pad), jnp.float32),
      mesh=mesh,
      compiler_params=pltpu.CompilerParams(needs_layout_passes=False),
      scratch_types=[
          pltpu.VMEM((2, half, _CHUNK), jnp.int32),
          pltpu.VMEM((n_pad,), jnp.float32),
          pltpu.SemaphoreType.DMA,
      ],
  )(dst3)


def _tc_body(a0_ref, a1_ref, degs_ref, x_ref, wl_ref, wr_ref, b_ref, o_ref):
  s = a0_ref[...] + a1_ref[...]
  deg = lax.dot_general(
      degs_ref[...], jnp.ones((_NW, 1), jnp.float32),
      (((0,), (0,)), ((), ())), preferred_element_type=jnp.float32)
  mean = s / jnp.maximum(deg, 1.0)
  acc = jnp.dot(mean, wl_ref[...], preferred_element_type=jnp.float32)
  acc = acc + jnp.dot(x_ref[...], wr_ref[...],
                      preferred_element_type=jnp.float32)
  o_ref[...] = jnp.maximum(acc + b_ref[...], 0.0)


def _tc_dense(a0, a1, degs, x, wlT, wrT, b, *, block_n):
  n, d = x.shape
  d_out = wlT.shape[1]
  grid = (-(-n // block_n),)
  return pl.pallas_call(
      _tc_body,
      grid=grid,
      in_specs=[
          pl.BlockSpec((block_n, d), lambda i: (i, 0)),
          pl.BlockSpec((block_n, d), lambda i: (i, 0)),
          pl.BlockSpec((_NW, block_n), lambda i: (0, i)),
          pl.BlockSpec((block_n, d), lambda i: (i, 0)),
          pl.BlockSpec((d, d_out), lambda i: (0, 0)),
          pl.BlockSpec((d, d_out), lambda i: (0, 0)),
          pl.BlockSpec((1, d_out), lambda i: (0, 0)),
      ],
      out_specs=pl.BlockSpec((block_n, d_out), lambda i: (i, 0)),
      out_shape=jax.ShapeDtypeStruct((n, d_out), jnp.float32),
  )(a0, a1, degs, x, wlT, wrT, b)


def kernel(x, edge_index, W_l, b_l, W_r):
  n, d = x.shape
  e = edge_index.shape[1]
  kq = 8 * _NPHASE
  k_chunks = -(-e // (_NW * _CHUNK * kq)) * kq
  e_pad = _NW * k_chunks * _CHUNK
  n_pad = -(-(n + 1) // (_NS * 8)) * (_NS * 8)

  src = edge_index[0]
  dst = edge_index[1]
  pad = e_pad - e
  src3 = jnp.concatenate([src, jnp.zeros((pad,), jnp.int32)]).reshape(
      _NW, k_chunks, _CHUNK)
  dst3 = jnp.concatenate([dst, jnp.full((pad,), n, jnp.int32)]).reshape(
      _NW, k_chunks, _CHUNK)

  acc = _sc_aggregate(x, src3, dst3, n_pad=n_pad, d=d, k_chunks=k_chunks)
  degs = _sc_degree(dst3, n_pad=n_pad, k_chunks=k_chunks)
  return _tc_dense(acc[0], acc[1], degs, x, W_l.T, W_r.T, b_l.reshape(1, -1),
                   block_n=1024)


# speedup vs baseline: 5.7135x; 5.7135x over previous
import functools

import jax
import jax.numpy as jnp
from jax import lax
from jax.experimental import pallas as pl
from jax.experimental.pallas import tpu as pltpu
from jax.experimental.pallas import tpu_sc as plsc

_NC = 2
_NS = 16
_NW = _NC * _NS
_CHUNK = 128


def _sc_aggregate(x, src3, dst3, *, n_pad, d, k_chunks):
  def body(x_hbm, src_hbm, dst_hbm, acc_out, deg_out, idx_src, idx_dst, rows,
           hist, acc_sh, sem):
    cid = lax.axis_index("c")
    sid = lax.axis_index("s")
    wid = sid * _NC + cid
    ones16 = jnp.ones((16,), jnp.float32)
    zeros16 = jnp.zeros((16,), jnp.float32)

    pltpu.sync_copy(src_hbm.at[wid], idx_src)
    pltpu.sync_copy(dst_hbm.at[wid], idx_dst)

    def zbody(i, c):
      for j in range(d // 16):
        rows[i, pl.ds(j * 16, 16)] = zeros16
      return c

    lax.fori_loop(0, _CHUNK, zbody, 0)

    def hbody(i, c):
      hist[pl.ds(i * 16, 16)] = zeros16
      return c

    lax.fori_loop(0, n_pad // 16, hbody, 0)

    zpt = n_pad // _NS
    base = sid * zpt
    for t in range(zpt // _CHUNK):
      pltpu.sync_copy(rows, acc_sh.at[pl.ds(base + t * _CHUNK, _CHUNK)])
    plsc.subcore_barrier()

    def ebody(j, c):
      pltpu.async_copy(x_hbm.at[idx_src.at[j]], rows, sem).wait()
      pltpu.sync_copy(rows, acc_sh.at[idx_dst.at[j]], add=True)
      for t in range(_CHUNK // 16):
        v = idx_dst[j, pl.ds(t * 16, 16)]
        plsc.addupdate_scatter(hist, [v], ones16)
      return c

    lax.fori_loop(0, k_chunks, ebody, 0)
    plsc.subcore_barrier()

    cpt = n_pad // _NS
    pltpu.sync_copy(acc_sh.at[pl.ds(sid * cpt, cpt)],
                    acc_out.at[cid, pl.ds(sid * cpt, cpt)])
    pltpu.sync_copy(hist, deg_out.at[wid])

  mesh = plsc.VectorSubcoreMesh(core_axis_name="c", subcore_axis_name="s")
  return pl.kernel(
      body,
      out_type=(
          jax.ShapeDtypeStruct((_NC, n_pad, d), jnp.float32),
          jax.ShapeDtypeStruct((_NW, n_pad), jnp.float32),
      ),
      mesh=mesh,
      compiler_params=pltpu.CompilerParams(needs_layout_passes=False),
      scratch_types=[
          pltpu.VMEM((k_chunks, _CHUNK), jnp.int32),
          pltpu.VMEM((k_chunks, _CHUNK), jnp.int32),
          pltpu.VMEM((_CHUNK, d), jnp.float32),
          pltpu.VMEM((n_pad,), jnp.float32),
          pltpu.VMEM_SHARED((n_pad, d), jnp.float32),
          pltpu.SemaphoreType.DMA,
      ],
  )(x, src3, dst3)


def _tc_body(a0_ref, a1_ref, degs_ref, x_ref, wl_ref, wr_ref, b_ref, o_ref):
  s = a0_ref[...] + a1_ref[...]
  deg = lax.dot_general(
      degs_ref[...], jnp.ones((_NW, 1), jnp.float32),
      (((0,), (0,)), ((), ())), preferred_element_type=jnp.float32)
  mean = s / jnp.maximum(deg, 1.0)
  acc = jnp.dot(mean, wl_ref[...], preferred_element_type=jnp.float32)
  acc = acc + jnp.dot(x_ref[...], wr_ref[...],
                      preferred_element_type=jnp.float32)
  o_ref[...] = jnp.maximum(acc + b_ref[...], 0.0)


def _tc_dense(a0, a1, degs, x, wlT, wrT, b, *, block_n):
  n, d = x.shape
  d_out = wlT.shape[1]
  grid = (-(-n // block_n),)
  return pl.pallas_call(
      _tc_body,
      grid=grid,
      in_specs=[
          pl.BlockSpec((block_n, d), lambda i: (i, 0)),
          pl.BlockSpec((block_n, d), lambda i: (i, 0)),
          pl.BlockSpec((_NW, block_n), lambda i: (0, i)),
          pl.BlockSpec((block_n, d), lambda i: (i, 0)),
          pl.BlockSpec((d, d_out), lambda i: (0, 0)),
          pl.BlockSpec((d, d_out), lambda i: (0, 0)),
          pl.BlockSpec((1, d_out), lambda i: (0, 0)),
      ],
      out_specs=pl.BlockSpec((block_n, d_out), lambda i: (i, 0)),
      out_shape=jax.ShapeDtypeStruct((n, d_out), jnp.float32),
  )(a0, a1, degs, x, wlT, wrT, b)


def kernel(x, edge_index, W_l, b_l, W_r):
  n, d = x.shape
  e = edge_index.shape[1]
  k_chunks = -(-e // (_NW * _CHUNK * 8)) * 8
  e_pad = _NW * k_chunks * _CHUNK
  n_pad = -(-(n + 1) // (_NS * _CHUNK)) * (_NS * _CHUNK)

  src = edge_index[0]
  dst = edge_index[1]
  pad = e_pad - e
  src3 = jnp.concatenate([src, jnp.zeros((pad,), jnp.int32)]).reshape(
      _NW, k_chunks, _CHUNK)
  dst3 = jnp.concatenate([dst, jnp.full((pad,), n, jnp.int32)]).reshape(
      _NW, k_chunks, _CHUNK)

  acc, degs = _sc_aggregate(x, src3, dst3, n_pad=n_pad, d=d,
                            k_chunks=k_chunks)
  return _tc_dense(acc[0], acc[1], degs, x, W_l.T, W_r.T, b_l.reshape(1, -1),
                   block_n=1024)


# speedup vs baseline: 5.7265x; 1.0023x over previous
import functools

import jax
import jax.numpy as jnp
from jax import lax
from jax.experimental import pallas as pl
from jax.experimental.pallas import tpu as pltpu
from jax.experimental.pallas import tpu_sc as plsc

_NC = 2
_NS = 16
_NW = _NC * _NS
_CHUNK = 128


def _sc_aggregate(x, src3, dst3, *, n_pad, d, k_chunks):
  def body(x_hbm, src_hbm, dst_hbm, acc_out, deg_out, idx_src, idx_dst, rows,
           hist, acc_sh, sem):
    cid = lax.axis_index("c")
    sid = lax.axis_index("s")
    wid = sid * _NC + cid
    ones16 = jnp.ones((16,), jnp.float32)
    zeros16 = jnp.zeros((16,), jnp.float32)

    pltpu.sync_copy(src_hbm.at[wid], idx_src)
    pltpu.sync_copy(dst_hbm.at[wid], idx_dst)

    def zbody(i, c):
      for j in range(d // 16):
        rows[i, pl.ds(j * 16, 16)] = zeros16
      return c

    lax.fori_loop(0, _CHUNK, zbody, 0)

    def hbody(i, c):
      hist[pl.ds(i * 16, 16)] = zeros16
      return c

    lax.fori_loop(0, n_pad // 16, hbody, 0)

    zpt = n_pad // _NS
    base = sid * zpt
    for t in range(zpt // _CHUNK):
      pltpu.sync_copy(rows, acc_sh.at[pl.ds(base + t * _CHUNK, _CHUNK)])
    rem = zpt % _CHUNK
    if rem:
      pltpu.sync_copy(rows.at[pl.ds(0, rem)],
                      acc_sh.at[pl.ds(base + zpt - rem, rem)])
    plsc.subcore_barrier()

    def ebody(j, c):
      pltpu.async_copy(x_hbm.at[idx_src.at[j]], rows, sem).wait()
      pltpu.sync_copy(rows, acc_sh.at[idx_dst.at[j]], add=True)
      for t in range(_CHUNK // 16):
        v = idx_dst[j, pl.ds(t * 16, 16)]
        plsc.addupdate_scatter(hist, [v], ones16)
      return c

    lax.fori_loop(0, k_chunks, ebody, 0)
    plsc.subcore_barrier()

    cpt = n_pad // _NS
    pltpu.sync_copy(acc_sh.at[pl.ds(sid * cpt, cpt)],
                    acc_out.at[cid, pl.ds(sid * cpt, cpt)])
    pltpu.sync_copy(hist, deg_out.at[wid])

  mesh = plsc.VectorSubcoreMesh(core_axis_name="c", subcore_axis_name="s")
  return pl.kernel(
      body,
      out_type=(
          jax.ShapeDtypeStruct((_NC, n_pad, d), jnp.float32),
          jax.ShapeDtypeStruct((_NW, n_pad), jnp.float32),
      ),
      mesh=mesh,
      compiler_params=pltpu.CompilerParams(needs_layout_passes=False),
      scratch_types=[
          pltpu.VMEM((k_chunks, _CHUNK), jnp.int32),
          pltpu.VMEM((k_chunks, _CHUNK), jnp.int32),
          pltpu.VMEM((_CHUNK, d), jnp.float32),
          pltpu.VMEM((n_pad,), jnp.float32),
          pltpu.VMEM_SHARED((n_pad, d), jnp.float32),
          pltpu.SemaphoreType.DMA,
      ],
  )(x, src3, dst3)


def _tc_body(a0_ref, a1_ref, degs_ref, x_ref, wl_ref, wr_ref, b_ref, o_ref):
  s = a0_ref[...] + a1_ref[...]
  deg = lax.dot_general(
      degs_ref[...], jnp.ones((_NW, 1), jnp.float32),
      (((0,), (0,)), ((), ())), preferred_element_type=jnp.float32)
  mean = s / jnp.maximum(deg, 1.0)
  acc = jnp.dot(mean, wl_ref[...], preferred_element_type=jnp.float32)
  acc = acc + jnp.dot(x_ref[...], wr_ref[...],
                      preferred_element_type=jnp.float32)
  o_ref[...] = jnp.maximum(acc + b_ref[...], 0.0)


def _tc_dense(a0, a1, degs, x, wlT, wrT, b, *, block_n):
  n, d = x.shape
  d_out = wlT.shape[1]
  grid = (-(-n // block_n),)
  return pl.pallas_call(
      _tc_body,
      grid=grid,
      in_specs=[
          pl.BlockSpec((block_n, d), lambda i: (i, 0)),
          pl.BlockSpec((block_n, d), lambda i: (i, 0)),
          pl.BlockSpec((_NW, block_n), lambda i: (0, i)),
          pl.BlockSpec((block_n, d), lambda i: (i, 0)),
          pl.BlockSpec((d, d_out), lambda i: (0, 0)),
          pl.BlockSpec((d, d_out), lambda i: (0, 0)),
          pl.BlockSpec((1, d_out), lambda i: (0, 0)),
      ],
      out_specs=pl.BlockSpec((block_n, d_out), lambda i: (i, 0)),
      out_shape=jax.ShapeDtypeStruct((n, d_out), jnp.float32),
  )(a0, a1, degs, x, wlT, wrT, b)


def kernel(x, edge_index, W_l, b_l, W_r):
  n, d = x.shape
  e = edge_index.shape[1]
  k_chunks = -(-e // (_NW * _CHUNK * 8)) * 8
  e_pad = _NW * k_chunks * _CHUNK
  n_pad = -(-(n + 1) // (_NS * 8)) * (_NS * 8)

  src = edge_index[0]
  dst = edge_index[1]
  pad = e_pad - e
  src3 = jnp.concatenate([src, jnp.zeros((pad,), jnp.int32)]).reshape(
      _NW, k_chunks, _CHUNK)
  dst3 = jnp.concatenate([dst, jnp.full((pad,), n, jnp.int32)]).reshape(
      _NW, k_chunks, _CHUNK)

  acc, degs = _sc_aggregate(x, src3, dst3, n_pad=n_pad, d=d,
                            k_chunks=k_chunks)
  return _tc_dense(acc[0], acc[1], degs, x, W_l.T, W_r.T, b_l.reshape(1, -1),
                   block_n=1024)


# speedup vs baseline: 6.5940x; 1.1515x over previous
import functools

import jax
import jax.numpy as jnp
from jax import lax
from jax.experimental import pallas as pl
from jax.experimental.pallas import tpu as pltpu
from jax.experimental.pallas import tpu_sc as plsc

_NC = 2
_NS = 16
_NW = _NC * _NS
_CHUNK = 128


def _sc_aggregate(x, src3, dst3, *, n_pad, d, k_chunks):
  half = k_chunks // 2

  def body(x_hbm, src_hbm, dst_hbm, acc_out, idx_src, idx_dst, rows0, rows1,
           acc_sh, sem0, sem1):
    cid = lax.axis_index("c")
    sid = lax.axis_index("s")
    wid = sid * _NC + cid
    zeros16 = jnp.zeros((16,), jnp.float32)
    bufs = ((rows0, sem0), (rows1, sem1))

    def zbody(i, c):
      for j in range(d // 16):
        rows0[i, pl.ds(j * 16, 16)] = zeros16
      return c

    lax.fori_loop(0, _CHUNK, zbody, 0)

    zpt = n_pad // _NS
    base = sid * zpt
    for t in range(zpt // _CHUNK):
      pltpu.sync_copy(rows0, acc_sh.at[pl.ds(base + t * _CHUNK, _CHUNK)])
    rem = zpt % _CHUNK
    if rem:
      pltpu.sync_copy(rows0.at[pl.ds(0, rem)],
                      acc_sh.at[pl.ds(base + zpt - rem, rem)])
    plsc.subcore_barrier()

    for h in range(2):
      pltpu.sync_copy(src_hbm.at[wid, pl.ds(h * half, half)], idx_src)
      pltpu.sync_copy(dst_hbm.at[wid, pl.ds(h * half, half)], idx_dst)

      for b, (buf, sem) in enumerate(bufs):
        pltpu.async_copy(x_hbm.at[idx_src.at[b]], buf, sem)

      def pbody(g, c):
        for b, (buf, sem) in enumerate(bufs):
          t = g * 2 + b
          pltpu.make_async_copy(x_hbm.at[idx_src.at[t]], buf, sem).wait()
          pltpu.sync_copy(buf, acc_sh.at[idx_dst.at[t]], add=True)
          pltpu.async_copy(x_hbm.at[idx_src.at[t + 2]], buf, sem)
        return c

      lax.fori_loop(0, (half - 2) // 2, pbody, 0)
      for b, (buf, sem) in enumerate(bufs):
        t = half - 2 + b
        pltpu.make_async_copy(x_hbm.at[idx_src.at[t]], buf, sem).wait()
        pltpu.sync_copy(buf, acc_sh.at[idx_dst.at[t]], add=True)

    plsc.subcore_barrier()

    cpt = n_pad // _NS
    pltpu.sync_copy(acc_sh.at[pl.ds(sid * cpt, cpt)],
                    acc_out.at[cid, pl.ds(sid * cpt, cpt)])

  mesh = plsc.VectorSubcoreMesh(core_axis_name="c", subcore_axis_name="s")
  return pl.kernel(
      body,
      out_type=jax.ShapeDtypeStruct((_NC, n_pad, d), jnp.float32),
      mesh=mesh,
      compiler_params=pltpu.CompilerParams(needs_layout_passes=False),
      scratch_types=[
          pltpu.VMEM((half, _CHUNK), jnp.int32),
          pltpu.VMEM((half, _CHUNK), jnp.int32),
          pltpu.VMEM((_CHUNK, d), jnp.float32),
          pltpu.VMEM((_CHUNK, d), jnp.float32),
          pltpu.VMEM_SHARED((n_pad, d), jnp.float32),
          pltpu.SemaphoreType.DMA,
          pltpu.SemaphoreType.DMA,
      ],
  )(x, src3, dst3)


def _sc_degree(dst3, *, n_pad, k_chunks):
  half = k_chunks // 2

  def body(dst_hbm, deg_out, idx, hist, isem):
    cid = lax.axis_index("c")
    sid = lax.axis_index("s")
    wid = sid * _NC + cid
    ones16 = jnp.ones((16,), jnp.float32)
    zeros16 = jnp.zeros((16,), jnp.float32)

    pltpu.sync_copy(dst_hbm.at[wid, pl.ds(0, half)], idx.at[0])
    pltpu.async_copy(dst_hbm.at[wid, pl.ds(half, half)], idx.at[1], isem)

    def hzero(i, c):
      hist[pl.ds(i * 16, 16)] = zeros16
      return c

    lax.fori_loop(0, n_pad // 16, hzero, 0)

    def count(s):
      def cbody(t, c):
        for q in range(_CHUNK // 16):
          v = idx[s, t, pl.ds(q * 16, 16)]
          plsc.addupdate_scatter(hist, [v], ones16)
        return c

      lax.fori_loop(0, half, cbody, 0)

    count(0)
    pltpu.make_async_copy(dst_hbm.at[wid, pl.ds(0, half)], idx.at[1],
                          isem).wait()
    count(1)
    pltpu.sync_copy(hist, deg_out.at[wid])

  mesh = plsc.VectorSubcoreMesh(core_axis_name="c", subcore_axis_name="s")
  return pl.kernel(
      body,
      out_type=jax.ShapeDtypeStruct((_NW, n_pad), jnp.float32),
      mesh=mesh,
      compiler_params=pltpu.CompilerParams(needs_layout_passes=False),
      scratch_types=[
          pltpu.VMEM((2, half, _CHUNK), jnp.int32),
          pltpu.VMEM((n_pad,), jnp.float32),
          pltpu.SemaphoreType.DMA,
      ],
  )(dst3)


def _tc_body(a0_ref, a1_ref, degs_ref, x_ref, wl_ref, wr_ref, b_ref, o_ref):
  s = a0_ref[...] + a1_ref[...]
  deg = lax.dot_general(
      degs_ref[...], jnp.ones((_NW, 1), jnp.float32),
      (((0,), (0,)), ((), ())), preferred_element_type=jnp.float32)
  mean = s / jnp.maximum(deg, 1.0)
  acc = jnp.dot(mean, wl_ref[...], preferred_element_type=jnp.float32)
  acc = acc + jnp.dot(x_ref[...], wr_ref[...],
                      preferred_element_type=jnp.float32)
  o_ref[...] = jnp.maximum(acc + b_ref[...], 0.0)


def _tc_dense(a0, a1, degs, x, wlT, wrT, b, *, block_n):
  n, d = x.shape
  d_out = wlT.shape[1]
  grid = (-(-n // block_n),)
  return pl.pallas_call(
      _tc_body,
      grid=grid,
      in_specs=[
          pl.BlockSpec((block_n, d), lambda i: (i, 0)),
          pl.BlockSpec((block_n, d), lambda i: (i, 0)),
          pl.BlockSpec((_NW, block_n), lambda i: (0, i)),
          pl.BlockSpec((block_n, d), lambda i: (i, 0)),
          pl.BlockSpec((d, d_out), lambda i: (0, 0)),
          pl.BlockSpec((d, d_out), lambda i: (0, 0)),
          pl.BlockSpec((1, d_out), lambda i: (0, 0)),
      ],
      out_specs=pl.BlockSpec((block_n, d_out), lambda i: (i, 0)),
      out_shape=jax.ShapeDtypeStruct((n, d_out), jnp.float32),
  )(a0, a1, degs, x, wlT, wrT, b)


def kernel(x, edge_index, W_l, b_l, W_r):
  n, d = x.shape
  e = edge_index.shape[1]
  k_chunks = -(-e // (_NW * _CHUNK * 16)) * 16
  e_pad = _NW * k_chunks * _CHUNK
  n_pad = -(-(n + 1) // (_NS * 8)) * (_NS * 8)

  src = edge_index[0]
  dst = edge_index[1]
  pad = e_pad - e
  src3 = jnp.concatenate([src, jnp.zeros((pad,), jnp.int32)]).reshape(
      _NW, k_chunks, _CHUNK)
  dst3 = jnp.concatenate([dst, jnp.full((pad,), n, jnp.int32)]).reshape(
      _NW, k_chunks, _CHUNK)

  acc = _sc_aggregate(x, src3, dst3, n_pad=n_pad, d=d, k_chunks=k_chunks)
  degs = _sc_degree(dst3, n_pad=n_pad, k_chunks=k_chunks)
  return _tc_dense(acc[0], acc[1], degs, x, W_l.T, W_r.T, b_l.reshape(1, -1),
                   block_n=1024)
